# baseline (speedup 1.0000x reference)
import dataclasses
import functools

import jax
import jax.numpy as jnp
from jax import lax
from jax.experimental import pallas as pl
from jax.experimental.pallas import tpu as pltpu
from jax.experimental.pallas import tpu_sc as plsc

N_V = 10000
M_E = 2000
NNZ = 160000
VP = 10240
ME_P = 2048
CHUNK = 4000
NCH = NNZ // CHUNK
LANES = 16
IPAD = 8
NC, NS = 2, 16


def _matmul_body(relu_in, wt_ref, b_ref, x_ref, o_ref):
    x = x_ref[...]
    if relu_in:
        x = jnp.maximum(x, 0.0)
    o_ref[...] = (
        jnp.dot(wt_ref[...], x, preferred_element_type=jnp.float32) + b_ref[...]
    )


def _tc_matmul(wt, b, xt, relu_in):
    co = wt.shape[0]
    return pl.pallas_call(
        functools.partial(_matmul_body, relu_in),
        out_shape=jax.ShapeDtypeStruct((co, VP), jnp.float32),
    )(wt, b.reshape(co, 1), xt)


def _sc_layer_body(cpt, last, x_hbm, vid_hbm, eid_hbm, out_hbm,
                   xs, es, deg, vbuf0, vbuf1, ebuf0, ebuf1, sems):
    vbufs = (vbuf0, vbuf1)
    ebufs = (ebuf0, ebuf1)
    wid = lax.axis_index("s") * NC + lax.axis_index("c")
    c0 = pl.multiple_of(wid * cpt, cpt)
    for cc in range(cpt):
        pltpu.sync_copy(x_hbm.at[c0 + cc], xs.at[pl.ds(cc * VP, VP)])

    zeros = jnp.zeros((LANES,), jnp.float32)
    ones = jnp.ones((LANES,), jnp.float32)

    def issue(ci, slot):
        off = pl.multiple_of(ci * CHUNK, 8)
        pltpu.async_copy(vid_hbm.at[pl.ds(off, CHUNK)],
                         vbufs[slot].at[pl.ds(IPAD, CHUNK)], sems.at[2 * slot])
        pltpu.async_copy(eid_hbm.at[pl.ds(off, CHUNK)],
                         ebufs[slot].at[pl.ds(IPAD, CHUNK)],
                         sems.at[2 * slot + 1])

    def wait(slot):
        pltpu.make_async_copy(vid_hbm.at[pl.ds(0, CHUNK)],
                              vbufs[slot].at[pl.ds(IPAD, CHUNK)],
                              sems.at[2 * slot]).wait()
        pltpu.make_async_copy(eid_hbm.at[pl.ds(0, CHUNK)],
                              ebufs[slot].at[pl.ds(IPAD, CHUNK)],
                              sems.at[2 * slot + 1]).wait()

    def sweep(process):
        issue(0, 0)
        issue(1, 1)

        @pl.loop(0, NCH - 2, step=2)
        def _(ci):
            wait(0)
            process(0)
            issue(ci + 2, 0)
            wait(1)
            process(1)
            issue(ci + 3, 1)

        wait(0)
        process(0)
        wait(1)
        process(1)

    @pl.loop(0, ME_P, step=LANES)
    def _(j):
        deg[pl.ds(j, LANES)] = zeros
        for cc in range(cpt):
            es[pl.ds(cc * ME_P + j, LANES)] = zeros

    laneidx = lax.iota(jnp.int32, LANES)
    lastlane = laneidx == (LANES - 1)
    notlast = laneidx < (LANES - 1)
    firstlane = laneidx == 0
    notfirst = laneidx > 0
    lanep1f = (laneidx + 1).astype(jnp.float32)
    nlanep1f = -lanep1f

    def pass1(slot):
        @plsc.parallel_loop(0, CHUNK, step=LANES, unroll=5)
        def _(i):
            v = vbufs[slot][pl.ds(i + IPAD, LANES)]
            e = ebufs[slot][pl.ds(i + IPAD, LANES)]
            en = ebufs[slot][pl.ds(i + IPAD + 1, LANES)]
            diff = e != en
            m_end = diff | lastlane
            m_mid = diff & notlast
            plsc.addupdate_scatter(deg, [e], lanep1f, mask=m_end)
            plsc.addupdate_scatter(deg, [en], nlanep1f, mask=m_mid)
            for cc in range(cpt):
                xv = plsc.load_gather(xs, [v + cc * VP])
                s = plsc.cumsum(xv)
                plsc.addupdate_scatter(es, [e + cc * ME_P], s, mask=m_end)
                plsc.addupdate_scatter(es, [en + cc * ME_P], -s, mask=m_mid)

    sweep(pass1)

    @pl.loop(0, ME_P, step=LANES)
    def _(j):
        dinv = 1.0 / jnp.maximum(deg[pl.ds(j, LANES)], 1.0)
        for cc in range(cpt):
            sl = pl.ds(cc * ME_P + j, LANES)
            es[sl] = es[sl] * dinv

    def pass3(slot):
        @plsc.parallel_loop(0, CHUNK, step=LANES, unroll=5)
        def _(i):
            v = vbufs[slot][pl.ds(i + IPAD, LANES)]
            e = ebufs[slot][pl.ds(i + IPAD, LANES)]
            for cc in range(cpt):
                yv = plsc.load_gather(es, [e + cc * ME_P])
                plsc.addupdate_scatter(xs, [v + cc * VP], yv)

    sweep(pass3)

    if last:
        @pl.loop(0, cpt * VP, step=LANES)
        def _(j):
            x = xs[pl.ds(j, LANES)]
            xs[pl.ds(j, LANES)] = 1.0 / (1.0 + jnp.exp(-x))

    for cc in range(cpt):
        pltpu.sync_copy(xs.at[pl.ds(cc * VP, VP)], out_hbm.at[c0 + cc])


def _sc_layer(xt, vid, eid, last):
    co = xt.shape[0]
    cpt = co // (NC * NS)
    mesh = plsc.VectorSubcoreMesh(core_axis_name="c", subcore_axis_name="s",
                                  num_cores=NC, num_subcores=NS)
    cp = pltpu.CompilerParams()
    if "needs_layout_passes" in pltpu.CompilerParams.__dataclass_fields__:
        cp = dataclasses.replace(cp, needs_layout_passes=False)
    return pl.kernel(
        functools.partial(_sc_layer_body, cpt, last),
        out_type=jax.ShapeDtypeStruct((co, VP), jnp.float32),
        mesh=mesh,
        compiler_params=cp,
        scratch_types=[
            pltpu.VMEM((cpt * VP,), jnp.float32),
            pltpu.VMEM((cpt * ME_P,), jnp.float32),
            pltpu.VMEM((ME_P,), jnp.float32),
            pltpu.VMEM((CHUNK + 2 * IPAD,), jnp.int32),
            pltpu.VMEM((CHUNK + 2 * IPAD,), jnp.int32),
            pltpu.VMEM((CHUNK + 2 * IPAD,), jnp.int32),
            pltpu.VMEM((CHUNK + 2 * IPAD,), jnp.int32),
            pltpu.SemaphoreType.DMA((4,)),
        ],
    )(xt, vid, eid)


def kernel(X, vertex_ids, edge_ids, W0, b0, W1, b1, W2, b2, W3, b3):
    xt = jnp.pad(X.T, ((0, 0), (0, VP - N_V)))
    for i, (w, b) in enumerate(((W0, b0), (W1, b1), (W2, b2), (W3, b3))):
        xt = _tc_matmul(w.T, b, xt, relu_in=(i > 0))
        xt = _sc_layer(xt, vertex_ids, edge_ids, last=(i == 3))
    return xt[:, :N_V].T


# speedup vs baseline: 1.1029x; 1.1029x over previous
import dataclasses
import functools

import jax
import jax.numpy as jnp
from jax import lax
from jax.experimental import pallas as pl
from jax.experimental.pallas import tpu as pltpu
from jax.experimental.pallas import tpu_sc as plsc

N_V = 10000
M_E = 2000
NNZ = 160000
VP = 10240
ME_P = 2048
CHUNK = 4000
NCH = NNZ // CHUNK
LANES = 16
IPAD = 8
NC, NS = 2, 16


def _matmul_body(relu_in, wt_ref, b_ref, x_ref, o_ref):
    x = x_ref[...]
    if relu_in:
        x = jnp.maximum(x, 0.0)
    o_ref[...] = (
        jnp.dot(wt_ref[...], x, preferred_element_type=jnp.float32) + b_ref[...]
    )


def _tc_matmul(wt, b, xt, relu_in):
    co = wt.shape[0]
    return pl.pallas_call(
        functools.partial(_matmul_body, relu_in),
        out_shape=jax.ShapeDtypeStruct((co, VP), jnp.float32),
    )(wt, b.reshape(co, 1), xt)


def _sc_layer_body(cpt, first, last, x_hbm, vid_hbm, eid_hbm, dinv_hbm,
                   out_hbm, dinv_out_hbm,
                   xs, es, deg, vbuf0, vbuf1, ebuf0, ebuf1, sems):
    vbufs = (vbuf0, vbuf1)
    ebufs = (ebuf0, ebuf1)
    wid = lax.axis_index("s") * NC + lax.axis_index("c")
    c0 = pl.multiple_of(wid * cpt, cpt)
    for cc in range(cpt):
        pltpu.sync_copy(x_hbm.at[c0 + cc], xs.at[pl.ds(cc * VP, VP)])
    if not first:
        pltpu.sync_copy(dinv_hbm, deg)

    zeros = jnp.zeros((LANES,), jnp.float32)
    ones = jnp.ones((LANES,), jnp.float32)

    def issue(ci, slot):
        off = pl.multiple_of(ci * CHUNK, 8)
        pltpu.async_copy(vid_hbm.at[pl.ds(off, CHUNK)],
                         vbufs[slot].at[pl.ds(IPAD, CHUNK)], sems.at[2 * slot])
        pltpu.async_copy(eid_hbm.at[pl.ds(off, CHUNK)],
                         ebufs[slot].at[pl.ds(IPAD, CHUNK)],
                         sems.at[2 * slot + 1])

    def wait(slot):
        pltpu.make_async_copy(vid_hbm.at[pl.ds(0, CHUNK)],
                              vbufs[slot].at[pl.ds(IPAD, CHUNK)],
                              sems.at[2 * slot]).wait()
        pltpu.make_async_copy(eid_hbm.at[pl.ds(0, CHUNK)],
                              ebufs[slot].at[pl.ds(IPAD, CHUNK)],
                              sems.at[2 * slot + 1]).wait()

    def sweep(process):
        issue(0, 0)
        issue(1, 1)

        @pl.loop(0, NCH - 2, step=2)
        def _(ci):
            wait(0)
            process(0)
            issue(ci + 2, 0)
            wait(1)
            process(1)
            issue(ci + 3, 1)

        wait(0)
        process(0)
        wait(1)
        process(1)

    @pl.loop(0, ME_P, step=LANES)
    def _(j):
        if first:
            deg[pl.ds(j, LANES)] = zeros
        for cc in range(cpt):
            es[pl.ds(cc * ME_P + j, LANES)] = zeros

    laneidx = lax.iota(jnp.int32, LANES)
    lastlane = laneidx == (LANES - 1)
    notlast = laneidx < (LANES - 1)
    firstlane = laneidx == 0
    notfirst = laneidx > 0
    lanep1f = (laneidx + 1).astype(jnp.float32)
    nlanep1f = -lanep1f

    def pass1(slot):
        @plsc.parallel_loop(0, CHUNK, step=LANES, unroll=5)
        def _(i):
            v = vbufs[slot][pl.ds(i + IPAD, LANES)]
            e = ebufs[slot][pl.ds(i + IPAD, LANES)]
            en = ebufs[slot][pl.ds(i + IPAD + 1, LANES)]
            diff = e != en
            m_end = diff | lastlane
            m_mid = diff & notlast
            if first:
                plsc.addupdate_scatter(deg, [e], lanep1f, mask=m_end)
                plsc.addupdate_scatter(deg, [en], nlanep1f, mask=m_mid)
            for cc in range(cpt):
                xv = plsc.load_gather(xs, [v + cc * VP])
                s = plsc.cumsum(xv)
                plsc.addupdate_scatter(es, [e + cc * ME_P], s, mask=m_end)
                plsc.addupdate_scatter(es, [en + cc * ME_P], -s, mask=m_mid)

    sweep(pass1)

    @pl.loop(0, ME_P, step=LANES)
    def _(j):
        if first:
            dinv = 1.0 / jnp.maximum(deg[pl.ds(j, LANES)], 1.0)
            deg[pl.ds(j, LANES)] = dinv
        else:
            dinv = deg[pl.ds(j, LANES)]
        for cc in range(cpt):
            sl = pl.ds(cc * ME_P + j, LANES)
            es[sl] = es[sl] * dinv

    if first:
        @pl.when(wid == 0)
        def _():
            pltpu.sync_copy(deg, dinv_out_hbm)

    def pass3(slot):
        @plsc.parallel_loop(0, CHUNK, step=LANES, unroll=5)
        def _(i):
            v = vbufs[slot][pl.ds(i + IPAD, LANES)]
            e = ebufs[slot][pl.ds(i + IPAD, LANES)]
            for cc in range(cpt):
                yv = plsc.load_gather(es, [e + cc * ME_P])
                plsc.addupdate_scatter(xs, [v + cc * VP], yv)

    sweep(pass3)

    if last:
        @pl.loop(0, cpt * VP, step=LANES)
        def _(j):
            x = xs[pl.ds(j, LANES)]
            xs[pl.ds(j, LANES)] = 1.0 / (1.0 + jnp.exp(-x))

    for cc in range(cpt):
        pltpu.sync_copy(xs.at[pl.ds(cc * VP, VP)], out_hbm.at[c0 + cc])


def _body_first(cpt, last, x, vid, eid, out, dinv_out, *scr):
    _sc_layer_body(cpt, True, last, x, vid, eid, None, out, dinv_out, *scr)


def _body_rest(cpt, last, x, vid, eid, dinv, out, *scr):
    _sc_layer_body(cpt, False, last, x, vid, eid, dinv, out, None, *scr)


def _sc_layer(xt, vid, eid, dinv, first, last):
    co = xt.shape[0]
    cpt = co // (NC * NS)
    mesh = plsc.VectorSubcoreMesh(core_axis_name="c", subcore_axis_name="s",
                                  num_cores=NC, num_subcores=NS)
    cp = pltpu.CompilerParams()
    if "needs_layout_passes" in pltpu.CompilerParams.__dataclass_fields__:
        cp = dataclasses.replace(cp, needs_layout_passes=False)
    out_t = jax.ShapeDtypeStruct((co, VP), jnp.float32)
    dinv_t = jax.ShapeDtypeStruct((ME_P,), jnp.float32)
    body = functools.partial(_body_first if first else _body_rest, cpt, last)
    k = pl.kernel(
        body,
        out_type=(out_t, dinv_t) if first else out_t,
        mesh=mesh,
        compiler_params=cp,
        scratch_types=[
            pltpu.VMEM((cpt * VP,), jnp.float32),
            pltpu.VMEM((cpt * ME_P,), jnp.float32),
            pltpu.VMEM((ME_P,), jnp.float32),
            pltpu.VMEM((CHUNK + 2 * IPAD,), jnp.int32),
            pltpu.VMEM((CHUNK + 2 * IPAD,), jnp.int32),
            pltpu.VMEM((CHUNK + 2 * IPAD,), jnp.int32),
            pltpu.VMEM((CHUNK + 2 * IPAD,), jnp.int32),
            pltpu.SemaphoreType.DMA((4,)),
        ],
    )
    if first:
        return k(xt, vid, eid)
    return k(xt, vid, eid, dinv)


def kernel(X, vertex_ids, edge_ids, W0, b0, W1, b1, W2, b2, W3, b3):
    xt = jnp.pad(X.T, ((0, 0), (0, VP - N_V)))
    dinv = None
    for i, (w, b) in enumerate(((W0, b0), (W1, b1), (W2, b2), (W3, b3))):
        xt = _tc_matmul(w.T, b, xt, relu_in=(i > 0))
        if i == 0:
            xt, dinv = _sc_layer(xt, vertex_ids, edge_ids, None,
                                 first=True, last=False)
        else:
            xt = _sc_layer(xt, vertex_ids, edge_ids, dinv,
                           first=False, last=(i == 3))
    return xt[:, :N_V].T


# speedup vs baseline: 1.1060x; 1.0028x over previous
import dataclasses
import functools

import jax
import jax.numpy as jnp
from jax import lax
from jax.experimental import pallas as pl
from jax.experimental.pallas import tpu as pltpu
from jax.experimental.pallas import tpu_sc as plsc

N_V = 10000
M_E = 2000
NNZ = 160000
VP = 10240
ME_P = 2048
CHUNK = 4000
NCH = NNZ // CHUNK
LANES = 16
IPAD = 8
NC, NS = 2, 16


def _matmul_body(relu_in, wt_ref, b_ref, x_ref, o_ref):
    x = x_ref[...]
    if relu_in:
        x = jnp.maximum(x, 0.0)
    o_ref[...] = (
        jnp.dot(wt_ref[...].astype(jnp.bfloat16), x.astype(jnp.bfloat16),
                preferred_element_type=jnp.float32) + b_ref[...]
    )


def _tc_matmul(wt, b, xt, relu_in):
    co, ci = wt.shape
    vb = 1280
    return pl.pallas_call(
        functools.partial(_matmul_body, relu_in),
        grid=(VP // vb,),
        in_specs=[
            pl.BlockSpec((co, ci), lambda i: (0, 0)),
            pl.BlockSpec((co, 1), lambda i: (0, 0)),
            pl.BlockSpec((ci, vb), lambda i: (0, i)),
        ],
        out_specs=pl.BlockSpec((co, vb), lambda i: (0, i)),
        out_shape=jax.ShapeDtypeStruct((co, VP), jnp.float32),
    )(wt, b.reshape(co, 1), xt)


def _sc_layer_body(cpt, first, last, x_hbm, vid_hbm, eid_hbm, dinv_hbm,
                   out_hbm, dinv_out_hbm,
                   xs, es, deg, vbuf0, vbuf1, ebuf0, ebuf1, sems):
    vbufs = (vbuf0, vbuf1)
    ebufs = (ebuf0, ebuf1)
    wid = lax.axis_index("s") * NC + lax.axis_index("c")
    c0 = pl.multiple_of(wid * cpt, cpt)

    def issue(ci, slot):
        off = pl.multiple_of(ci * CHUNK, 8)
        pltpu.async_copy(vid_hbm.at[pl.ds(off, CHUNK)],
                         vbufs[slot].at[pl.ds(IPAD, CHUNK)], sems.at[2 * slot])
        pltpu.async_copy(eid_hbm.at[pl.ds(off, CHUNK)],
                         ebufs[slot].at[pl.ds(IPAD, CHUNK)],
                         sems.at[2 * slot + 1])

    xcps = [pltpu.make_async_copy(x_hbm.at[c0 + cc],
                                  xs.at[pl.ds(cc * VP, VP)], sems.at[4])
            for cc in range(cpt)]
    for c in xcps:
        c.start()
    issue(0, 0)
    issue(1, 1)
    if not first:
        pltpu.sync_copy(dinv_hbm, deg)
    for c in xcps:
        c.wait()

    zeros = jnp.zeros((LANES,), jnp.float32)
    ones = jnp.ones((LANES,), jnp.float32)

    def wait(slot):
        pltpu.make_async_copy(vid_hbm.at[pl.ds(0, CHUNK)],
                              vbufs[slot].at[pl.ds(IPAD, CHUNK)],
                              sems.at[2 * slot]).wait()
        pltpu.make_async_copy(eid_hbm.at[pl.ds(0, CHUNK)],
                              ebufs[slot].at[pl.ds(IPAD, CHUNK)],
                              sems.at[2 * slot + 1]).wait()

    def sweep(process, pre_issued=False):
        if not pre_issued:
            issue(0, 0)
            issue(1, 1)

        @pl.loop(0, NCH - 2, step=2)
        def _(ci):
            wait(0)
            process(0)
            issue(ci + 2, 0)
            wait(1)
            process(1)
            issue(ci + 3, 1)

        wait(0)
        process(0)
        wait(1)
        process(1)

    @pl.loop(0, ME_P, step=LANES)
    def _(j):
        if first:
            deg[pl.ds(j, LANES)] = zeros
        for cc in range(cpt):
            es[pl.ds(cc * ME_P + j, LANES)] = zeros

    laneidx = lax.iota(jnp.int32, LANES)
    lastlane = laneidx == (LANES - 1)
    notlast = laneidx < (LANES - 1)
    firstlane = laneidx == 0
    notfirst = laneidx > 0
    lanep1f = (laneidx + 1).astype(jnp.float32)
    nlanep1f = -lanep1f

    def pass1(slot):
        @plsc.parallel_loop(0, CHUNK, step=LANES, unroll=5)
        def _(i):
            v = vbufs[slot][pl.ds(i + IPAD, LANES)]
            e = ebufs[slot][pl.ds(i + IPAD, LANES)]
            en = ebufs[slot][pl.ds(i + IPAD + 1, LANES)]
            diff = e != en
            m_end = diff | lastlane
            m_mid = diff & notlast
            if first:
                plsc.addupdate_scatter(deg, [e], lanep1f, mask=m_end)
                plsc.addupdate_scatter(deg, [en], nlanep1f, mask=m_mid)
            for cc in range(cpt):
                xv = plsc.load_gather(xs, [v + cc * VP])
                s = plsc.cumsum(xv)
                plsc.addupdate_scatter(es, [e + cc * ME_P], s, mask=m_end)
                plsc.addupdate_scatter(es, [en + cc * ME_P], -s, mask=m_mid)

    sweep(pass1, pre_issued=True)

    @pl.loop(0, ME_P, step=LANES)
    def _(j):
        if first:
            dinv = 1.0 / jnp.maximum(deg[pl.ds(j, LANES)], 1.0)
            deg[pl.ds(j, LANES)] = dinv
        else:
            dinv = deg[pl.ds(j, LANES)]
        for cc in range(cpt):
            sl = pl.ds(cc * ME_P + j, LANES)
            es[sl] = es[sl] * dinv

    if first:
        @pl.when(wid == 0)
        def _():
            pltpu.sync_copy(deg, dinv_out_hbm)

    def pass3(slot):
        @plsc.parallel_loop(0, CHUNK, step=LANES, unroll=5)
        def _(i):
            v = vbufs[slot][pl.ds(i + IPAD, LANES)]
            e = ebufs[slot][pl.ds(i + IPAD, LANES)]
            for cc in range(cpt):
                yv = plsc.load_gather(es, [e + cc * ME_P])
                plsc.addupdate_scatter(xs, [v + cc * VP], yv)

    sweep(pass3)

    if last:
        @pl.loop(0, cpt * VP, step=LANES)
        def _(j):
            x = xs[pl.ds(j, LANES)]
            xs[pl.ds(j, LANES)] = 1.0 / (1.0 + jnp.exp(-x))

    ocps = [pltpu.make_async_copy(xs.at[pl.ds(cc * VP, VP)],
                                  out_hbm.at[c0 + cc], sems.at[4])
            for cc in range(cpt)]
    for c in ocps:
        c.start()
    for c in ocps:
        c.wait()


def _body_first(cpt, last, x, vid, eid, out, dinv_out, *scr):
    _sc_layer_body(cpt, True, last, x, vid, eid, None, out, dinv_out, *scr)


def _body_rest(cpt, last, x, vid, eid, dinv, out, *scr):
    _sc_layer_body(cpt, False, last, x, vid, eid, dinv, out, None, *scr)


def _sc_layer(xt, vid, eid, dinv, first, last):
    co = xt.shape[0]
    cpt = co // (NC * NS)
    mesh = plsc.VectorSubcoreMesh(core_axis_name="c", subcore_axis_name="s",
                                  num_cores=NC, num_subcores=NS)
    cp = pltpu.CompilerParams()
    if "needs_layout_passes" in pltpu.CompilerParams.__dataclass_fields__:
        cp = dataclasses.replace(cp, needs_layout_passes=False)
    out_t = jax.ShapeDtypeStruct((co, VP), jnp.float32)
    dinv_t = jax.ShapeDtypeStruct((ME_P,), jnp.float32)
    body = functools.partial(_body_first if first else _body_rest, cpt, last)
    k = pl.kernel(
        body,
        out_type=(out_t, dinv_t) if first else out_t,
        mesh=mesh,
        compiler_params=cp,
        scratch_types=[
            pltpu.VMEM((cpt * VP,), jnp.float32),
            pltpu.VMEM((cpt * ME_P,), jnp.float32),
            pltpu.VMEM((ME_P,), jnp.float32),
            pltpu.VMEM((CHUNK + 2 * IPAD,), jnp.int32),
            pltpu.VMEM((CHUNK + 2 * IPAD,), jnp.int32),
            pltpu.VMEM((CHUNK + 2 * IPAD,), jnp.int32),
            pltpu.VMEM((CHUNK + 2 * IPAD,), jnp.int32),
            pltpu.SemaphoreType.DMA((5,)),
        ],
    )
    if first:
        return k(xt, vid, eid)
    return k(xt, vid, eid, dinv)


def kernel(X, vertex_ids, edge_ids, W0, b0, W1, b1, W2, b2, W3, b3):
    xt = jnp.pad(X.T, ((0, 0), (0, VP - N_V)))
    dinv = None
    for i, (w, b) in enumerate(((W0, b0), (W1, b1), (W2, b2), (W3, b3))):
        xt = _tc_matmul(w.T, b, xt, relu_in=(i > 0))
        if i == 0:
            xt, dinv = _sc_layer(xt, vertex_ids, edge_ids, None,
                                 first=True, last=False)
        else:
            xt = _sc_layer(xt, vertex_ids, edge_ids, dinv,
                           first=False, last=(i == 3))
    return xt[:, :N_V].T
